# Initial kernel scaffold; baseline (speedup 1.0000x reference)
import jax
import jax.numpy as jnp
from jax.experimental import pallas as pl


def kernel(inventory_levels, priorities, edge_index, W1, b1, W2, b2):
    raise NotImplementedError("write your pallas kernel here")


# speedup vs baseline: 5.2031x; 5.2031x over previous
import functools

import jax
import jax.numpy as jnp
from jax import lax
from jax.experimental import pallas as pl
from jax.experimental.pallas import tpu as pltpu
from jax.experimental.pallas import tpu_sc as plsc

N = 10000
NP = 10240
E = 320000
D1 = 128
D2 = 16
K = 125
G = 8
ER = E // K
NC = 2
NS = 16
RPS = NP // NS

_MESH = plsc.VectorSubcoreMesh(
    core_axis_name="c", subcore_axis_name="s", num_cores=NC, num_subcores=NS
)


ZR = 40


def _zero_rows(buf, nrows, d):
    per_row = d // 16

    def body(i, carry):
        r = i // per_row
        col = (i % per_row) * 16
        buf[r, pl.ds(col, 16)] = jnp.zeros((16,), jnp.float32)
        return carry

    lax.fori_loop(0, nrows * per_row, body, 0)


def _zero_acc_slice(zbuf, acc_sh, s, d):
    _zero_rows(zbuf, ZR, d)

    def body(t, carry):
        pltpu.sync_copy(zbuf, acc_sh.at[pl.ds(s * RPS + t * ZR, ZR)])
        return carry

    lax.fori_loop(0, RPS // ZR, body, 0)


HR = NP // 128
BD = 2560


def _deg_body(e2_ref, et_ref, out_ref):
    @pl.when(pl.program_id(0) == 0)
    def _():
        out_ref[...] = jnp.zeros_like(out_ref)

    rows = lax.broadcasted_iota(jnp.int32, (HR, BD), 0)
    cols = lax.broadcasted_iota(jnp.int32, (BD, 128), 1)
    for a in range(2):
        hi = e2_ref[a : a + 1, :] >> 7
        lo = et_ref[:, a : a + 1] & 127
        hi_oh = (rows == hi).astype(jnp.float32)
        lo_oh = (cols == lo).astype(jnp.float32)
        out_ref[a] += jnp.dot(
            hi_oh, lo_oh, preferred_element_type=jnp.float32
        )


_deg_kernel = pl.pallas_call(
    _deg_body,
    grid=(E // BD,),
    in_specs=[
        pl.BlockSpec((2, BD), lambda i: (0, i)),
        pl.BlockSpec((BD, 2), lambda i: (i, 0)),
    ],
    out_specs=pl.BlockSpec((2, HR, 128), lambda i: (0, 0, 0)),
    out_shape=jax.ShapeDtypeStruct((2, HR, 128), jnp.float32),
)


def _make_agg_kernel(d):
    def body(xw_hbm, er_hbm, out_hbm, sidx, didx, rows_v, zbuf, acc_sh,
             semg, sems):
        c = lax.axis_index("c")
        s = lax.axis_index("s")
        _zero_acc_slice(zbuf, acc_sh, s, d)
        plsc.subcore_barrier()

        groups = ER // (NC * NS * G)
        base = (c * NS + s) * groups

        def loop(g, carry):
            row = (base + g) * G
            pltpu.sync_copy(er_hbm.at[0, pl.ds(row, G)], sidx)
            pltpu.sync_copy(er_hbm.at[1, pl.ds(row, G)], didx)
            gd = [None] * G
            sd = [None] * G
            gd[0] = pltpu.async_copy(xw_hbm.at[sidx.at[0]], rows_v.at[0], semg)
            for j in range(G):
                gd[j].wait()
                if j >= 1:
                    sd[j - 1].wait()
                if j + 1 < G:
                    gd[j + 1] = pltpu.async_copy(
                        xw_hbm.at[sidx.at[j + 1]], rows_v.at[(j + 1) % 2], semg
                    )
                sd[j] = pltpu.async_copy(
                    rows_v.at[j % 2], acc_sh.at[didx.at[j]], sems, add=True
                )
            sd[G - 1].wait()
            return carry

        lax.fori_loop(0, groups, loop, 0)
        plsc.subcore_barrier()
        sl = pl.ds(s * RPS, RPS)
        pltpu.sync_copy(acc_sh.at[sl], out_hbm.at[c, sl])

    return functools.partial(
        pl.kernel,
        out_type=jax.ShapeDtypeStruct((2, NP, d), jnp.float32),
        mesh=_MESH,
        scratch_types=[
            pltpu.VMEM((G, K), jnp.int32),
            pltpu.VMEM((G, K), jnp.int32),
            pltpu.VMEM((2, K, d), jnp.float32),
            pltpu.VMEM((ZR, d), jnp.float32),
            pltpu.VMEM_SHARED((NP, d), jnp.float32),
            pltpu.SemaphoreType.DMA,
            pltpu.SemaphoreType.DMA,
        ],
    )(body)


_agg = _make_agg_kernel(D1)

_R = 640


def _tc_b_body(inv_ref, pri_ref, w1_ref, degs_ref, out_ref):
    x = jnp.concatenate([inv_ref[...], pri_ref[...]], axis=1)
    s_out = lax.rsqrt(jnp.maximum(degs_ref[:, 0:1], 1.0))
    out_ref[...] = jnp.dot(
        x * s_out, w1_ref[...], preferred_element_type=jnp.float32
    )


_tc_b = pl.pallas_call(
    _tc_b_body,
    grid=(NP // _R,),
    in_specs=[
        pl.BlockSpec((_R, 64), lambda i: (i, 0)),
        pl.BlockSpec((_R, 64), lambda i: (i, 0)),
        pl.BlockSpec((D1, D1), lambda i: (0, 0)),
        pl.BlockSpec((_R, 2), lambda i: (i, 0)),
    ],
    out_specs=pl.BlockSpec((_R, D1), lambda i: (i, 0)),
    out_shape=jax.ShapeDtypeStruct((NP, D1), jnp.float32),
)


def _tc_d_body(p_ref, degs_ref, b1_ref, out_ref):
    agg = p_ref[0] + p_ref[1]
    s_out = lax.rsqrt(jnp.maximum(degs_ref[:, 0:1], 1.0))
    s_in = lax.rsqrt(jnp.maximum(degs_ref[:, 1:2], 1.0))
    h = jnp.maximum(agg * s_in + b1_ref[...], 0.0)
    out_ref[...] = h * s_out


_tc_d = pl.pallas_call(
    _tc_d_body,
    grid=(NP // _R,),
    in_specs=[
        pl.BlockSpec((2, _R, D1), lambda i: (0, i, 0)),
        pl.BlockSpec((_R, 2), lambda i: (i, 0)),
        pl.BlockSpec((1, D1), lambda i: (0, 0)),
    ],
    out_specs=pl.BlockSpec((_R, D1), lambda i: (i, 0)),
    out_shape=jax.ShapeDtypeStruct((NP, D1), jnp.float32),
)


def _tc_f_body(p_ref, degs_ref, w2_ref, b2_ref, out_ref):
    agg = p_ref[0] + p_ref[1]
    s_in = lax.rsqrt(jnp.maximum(degs_ref[:, 1:2], 1.0))
    out_ref[...] = (
        jnp.dot(agg * s_in, w2_ref[...], preferred_element_type=jnp.float32)
        + b2_ref[...]
    )


_tc_f = pl.pallas_call(
    _tc_f_body,
    grid=(NP // _R,),
    in_specs=[
        pl.BlockSpec((2, _R, D1), lambda i: (0, i, 0)),
        pl.BlockSpec((_R, 2), lambda i: (i, 0)),
        pl.BlockSpec((D1, D2), lambda i: (0, 0)),
        pl.BlockSpec((1, D2), lambda i: (0, 0)),
    ],
    out_specs=pl.BlockSpec((_R, D2), lambda i: (i, 0)),
    out_shape=jax.ShapeDtypeStruct((NP, D2), jnp.float32),
)


def kernel(inventory_levels, priorities, edge_index, W1, b1, W2, b2):
    pad = ((0, NP - N), (0, 0))
    inv_p = jnp.pad(inventory_levels, pad)
    pri_p = jnp.pad(priorities, pad)
    er = edge_index.reshape(2, ER, K)
    dcounts = _deg_kernel(edge_index, edge_index.T)
    degs = dcounts.reshape(2, NP).T
    xw1 = _tc_b(inv_p, pri_p, W1, degs)
    p1 = _agg(xw1, er)
    hs = _tc_d(p1, degs, b1.reshape(1, D1))
    p2 = _agg(hs, er)
    out = _tc_f(p2, degs, W2, b2.reshape(1, D2))
    return out[:N]
